# baseline (speedup 1.0000x reference)
import jax
import jax.numpy as jnp
from jax import lax
from jax.experimental import pallas as pl
from jax.experimental.pallas import tpu as pltpu
from jax.experimental.pallas import tpu_sc as plsc

NX = 10
NE = 5
T = 1000
N_NODES = 1024

_NW = 32
_ECHUNK = 1024


def _alpha(t):
    halfpi = 0.5 * jnp.pi
    s = 0.01
    return jnp.cos(halfpi * (t / T + s) / (1 + s))


def _recip(x):
    ax = jnp.abs(x)
    seed = lax.bitcast_convert_type(
        jnp.int32(0x7EF477D5) - lax.bitcast_convert_type(ax, jnp.int32),
        jnp.float32)
    r = seed * (2.0 - ax * seed)
    r = r * (2.0 - ax * r)
    r = r * (2.0 - ax * r)
    return jnp.where(x < 0.0, -r, r)


def _rows16(inref, outref, rb, ca, cb, o, d, C):
    riv = rb + lax.iota(jnp.int32, 16)
    rin = riv * C
    rout = riv * (C * C)
    cols = [plsc.load_gather(inref, [rin + j]) for j in range(C)]
    s = cols[0]
    for j in range(1, C):
        s = s + cols[j]
    L = [ca * s + cb * cols[j] for j in range(C)]
    r = [_recip(jnp.where(L[j] == 0.0, 1e-6, L[j])) for j in range(C)]
    to = [o * L[k] for k in range(C)]
    td = [d * L[k] for k in range(C)]
    for j in range(C):
        for k in range(C):
            val = r[j] * (td[k] if j == k else to[k])
            plsc.store_scatter(outref, [rout + (j * C + k)], val)


def _sc_body(Xt_hbm, Et_hbm, par_hbm, outX_hbm, outE_hbm,
             par_v, inX, outXb,
             inE0, inE1, outE0, outE1,
             isem0, isem1, osem0, osem1):
    wid = lax.axis_index("s") * 2 + lax.axis_index("c")

    in_bufs = (inE0, inE1)
    out_bufs = (outE0, outE1)
    in_sems = (isem0, isem1)
    out_sems = (osem0, osem1)

    erows = N_NODES * N_NODES // _NW
    nchunk = erows // _ECHUNK
    ebase = wid * erows

    def e_src(t):
        return Et_hbm.at[pl.ds((ebase + t * _ECHUNK) * NE, _ECHUNK * NE)]

    def e_dst(t):
        return outE_hbm.at[
            pl.ds((ebase + t * _ECHUNK) * NE * NE, _ECHUNK * NE * NE)]

    pltpu.make_async_copy(e_src(0), inE0, isem0).start()
    pltpu.make_async_copy(e_src(1), inE1, isem1).start()

    pltpu.sync_copy(par_hbm, par_v)
    caE, cbE, oE, dE = par_v[0], par_v[1], par_v[2], par_v[3]
    caX, cbX, oX, dX = par_v[4], par_v[5], par_v[6], par_v[7]

    xrows = N_NODES // _NW
    xbase = wid * xrows
    pltpu.sync_copy(Xt_hbm.at[pl.ds(xbase * NX, xrows * NX)], inX)
    for i in range(xrows // 16):
        _rows16(inX, outXb, i * 16, caX, cbX, oX, dX, NX)
    pltpu.sync_copy(outXb, outX_hbm.at[pl.ds(xbase * NX * NX,
                                             xrows * NX * NX)])

    def chunk_compute(inref, outref):
        def body(i, c):
            _rows16(inref, outref, i * 16, caE, cbE, oE, dE, NE)
            return c
        lax.fori_loop(0, _ECHUNK // 16, body, 0)

    def pair(p, c):
        for b in range(2):
            t = 2 * p + b
            pltpu.make_async_copy(e_src(0), in_bufs[b], in_sems[b]).wait()

            @pl.when(t >= 2)
            def _():
                pltpu.make_async_copy(
                    out_bufs[b], e_dst(0), out_sems[b]).wait()

            chunk_compute(in_bufs[b], out_bufs[b])

            @pl.when(t < nchunk - 2)
            def _():
                pltpu.make_async_copy(
                    e_src(t + 2), in_bufs[b], in_sems[b]).start()

            pltpu.make_async_copy(out_bufs[b], e_dst(t), out_sems[b]).start()
        return c

    lax.fori_loop(0, nchunk // 2, pair, 0)
    pltpu.make_async_copy(outE0, e_dst(0), osem0).wait()
    pltpu.make_async_copy(outE1, e_dst(1), osem1).wait()


def kernel(Xt, Et, t):
    if Et.ndim == 3:
        Et = Et.reshape((-1, Et.shape[-1]))
    n_edges = Et.shape[0]

    a = _alpha(t).astype(jnp.float32)
    ab = _alpha(t - 1).astype(jnp.float32)
    caE, cbE = a / NE, 1.0 - a
    oE = ab / NE
    dE = oE + (1.0 - ab)
    caX, cbX = a / NX, 1.0 - a
    oX = ab / NX
    dX = oX + (1.0 - ab)
    par = jnp.stack([caE, cbE, oE, dE, caX, cbX, oX, dX]).astype(jnp.float32)
    par = jnp.tile(par[:, None], (1, 16))

    mesh = plsc.VectorSubcoreMesh(core_axis_name="c", subcore_axis_name="s")
    run = pl.kernel(
        _sc_body,
        out_type=(
            jax.ShapeDtypeStruct((N_NODES * NX * NX,), jnp.float32),
            jax.ShapeDtypeStruct((n_edges * NE * NE,), jnp.float32),
        ),
        mesh=mesh,
        scratch_types=[
            pltpu.VMEM((8, 16), jnp.float32),
            pltpu.VMEM((N_NODES // _NW * NX,), jnp.float32),
            pltpu.VMEM((N_NODES // _NW * NX * NX,), jnp.float32),
            pltpu.VMEM((_ECHUNK * NE,), jnp.float32),
            pltpu.VMEM((_ECHUNK * NE,), jnp.float32),
            pltpu.VMEM((_ECHUNK * NE * NE,), jnp.float32),
            pltpu.VMEM((_ECHUNK * NE * NE,), jnp.float32),
            pltpu.SemaphoreType.DMA,
            pltpu.SemaphoreType.DMA,
            pltpu.SemaphoreType.DMA,
            pltpu.SemaphoreType.DMA,
        ],
        compiler_params=pltpu.CompilerParams(needs_layout_passes=False),
        name="discrete_noise_posterior_sc",
    )
    Xp, Ep = run(Xt.reshape(-1), Et.reshape(-1), par)
    return (Xp.reshape(N_NODES, NX, NX), Ep.reshape(n_edges, NE, NE))


# speedup vs baseline: 6.0755x; 6.0755x over previous
import jax
import jax.numpy as jnp
from jax.experimental import pallas as pl

NX = 10
NE = 5
T = 1000
N_NODES = 1024

_NB = 1024


def _alpha(t):
    halfpi = 0.5 * jnp.pi
    s = 0.01
    return jnp.cos(halfpi * (t / T + s) / (1 + s))


def _posterior_body(x_ref, q_ref, c_ref, o_ref):
    C = x_ref.shape[0]
    nb = x_ref.shape[1]
    ca = c_ref[0, 0]
    cb = c_ref[1, 0]
    x = x_ref[...]
    s = jnp.sum(x, axis=0, keepdims=True)
    L = ca * s + cb * x
    den = jnp.where(L == 0.0, 1e-6, L)
    R = 1.0 / den
    q = jnp.broadcast_to(q_ref[...], (C, C, nb))
    o_ref[...] = q * R[:, None, :] * L[None, :, :]


def _run(vT, q3, coef, C, n, nb):
    grid = n // nb
    return pl.pallas_call(
        _posterior_body,
        grid=(grid,),
        in_specs=[
            pl.BlockSpec((C, nb), lambda i: (0, i)),
            pl.BlockSpec((C, C, 1), lambda i: (0, 0, 0)),
            pl.BlockSpec((2, 1), lambda i: (0, 0)),
        ],
        out_specs=pl.BlockSpec((C, C, nb), lambda i: (0, 0, i)),
        out_shape=jax.ShapeDtypeStruct((C, C, n), jnp.float32),
        name=f"discrete_noise_posterior_c{C}",
    )(vT, q3, coef)


def kernel(Xt, Et, t):
    if Et.ndim == 3:
        Et = Et.reshape((-1, Et.shape[-1]))
    n_edges = Et.shape[0]

    a = _alpha(t).astype(jnp.float32)
    ab = _alpha(t - 1).astype(jnp.float32)

    def q3_of(C):
        o = ab / C
        d = o + (1.0 - ab)
        q = jnp.full((C, C), o, jnp.float32) + (d - o) * jnp.eye(
            C, dtype=jnp.float32)
        return q[:, :, None]

    def coef_of(C):
        return jnp.stack([a / C, 1.0 - a]).astype(jnp.float32)[:, None]

    Ep = _run(Et.T, q3_of(NE), coef_of(NE), NE, n_edges, _NB)
    Xp = _run(Xt.T, q3_of(NX), coef_of(NX), NX, N_NODES, N_NODES)
    return (jnp.transpose(Xp, (2, 0, 1)), jnp.transpose(Ep, (2, 0, 1)))


# speedup vs baseline: 17.7283x; 2.9180x over previous
import jax
import jax.numpy as jnp
from jax.experimental import pallas as pl

NX = 10
NE = 5
T = 1000
N_NODES = 1024

_NB = 4096


def _alpha(t):
    halfpi = 0.5 * jnp.pi
    s = 0.01
    return jnp.cos(halfpi * (t / T + s) / (1 + s))


def _posterior_body(x_ref, q_ref, c_ref, o_ref):
    C = x_ref.shape[0]
    nb = x_ref.shape[1]
    ca = c_ref[0, 0]
    cb = c_ref[1, 0]
    x = x_ref[...]
    s = jnp.sum(x, axis=0, keepdims=True)
    L = ca * s + cb * x
    den = jnp.where(L == 0.0, 1e-6, L)
    R = 1.0 / den
    q = jnp.broadcast_to(q_ref[...], (C, C, nb))
    o_ref[...] = q * R[:, None, :] * L[None, :, :]


def _run(vT, q3, coef, C, n, nb):
    grid = n // nb
    return pl.pallas_call(
        _posterior_body,
        grid=(grid,),
        in_specs=[
            pl.BlockSpec((C, nb), lambda i: (0, i)),
            pl.BlockSpec((C, C, 1), lambda i: (0, 0, 0)),
            pl.BlockSpec((2, 1), lambda i: (0, 0)),
        ],
        out_specs=pl.BlockSpec((C, C, nb), lambda i: (0, 0, i)),
        out_shape=jax.ShapeDtypeStruct((C, C, n), jnp.float32),
        name=f"discrete_noise_posterior_c{C}",
    )(vT, q3, coef)


def kernel(Xt, Et, t):
    if Et.ndim == 3:
        Et = Et.reshape((-1, Et.shape[-1]))
    n_edges = Et.shape[0]

    a = _alpha(t).astype(jnp.float32)
    ab = _alpha(t - 1).astype(jnp.float32)

    def q3_of(C):
        o = ab / C
        d = o + (1.0 - ab)
        q = jnp.full((C, C), o, jnp.float32) + (d - o) * jnp.eye(
            C, dtype=jnp.float32)
        return q[:, :, None]

    def coef_of(C):
        return jnp.stack([a / C, 1.0 - a]).astype(jnp.float32)[:, None]

    Ep = _run(Et.T, q3_of(NE), coef_of(NE), NE, n_edges, _NB)
    Xp = _run(Xt.T, q3_of(NX), coef_of(NX), NX, N_NODES, N_NODES)
    return (jnp.transpose(Xp, (2, 0, 1)), jnp.transpose(Ep, (2, 0, 1)))


# speedup vs baseline: 38.5611x; 2.1751x over previous
import jax
import jax.numpy as jnp
from jax.experimental import pallas as pl

NX = 10
NE = 5
T = 1000
N_NODES = 1024

_NB = 32768


def _alpha(t):
    halfpi = 0.5 * jnp.pi
    s = 0.01
    return jnp.cos(halfpi * (t / T + s) / (1 + s))


def _posterior_body(x_ref, q_ref, c_ref, o_ref):
    C = x_ref.shape[0]
    nb = x_ref.shape[1]
    ca = c_ref[0, 0]
    cb = c_ref[1, 0]
    x = x_ref[...]
    s = jnp.sum(x, axis=0, keepdims=True)
    L = ca * s + cb * x
    den = jnp.where(L == 0.0, 1e-6, L)
    R = 1.0 / den
    q = jnp.broadcast_to(q_ref[...], (C, C, nb))
    o_ref[...] = q * R[:, None, :] * L[None, :, :]


def _run(vT, q3, coef, C, n, nb):
    grid = n // nb
    return pl.pallas_call(
        _posterior_body,
        grid=(grid,),
        in_specs=[
            pl.BlockSpec((C, nb), lambda i: (0, i)),
            pl.BlockSpec((C, C, 1), lambda i: (0, 0, 0)),
            pl.BlockSpec((2, 1), lambda i: (0, 0)),
        ],
        out_specs=pl.BlockSpec((C, C, nb), lambda i: (0, 0, i)),
        out_shape=jax.ShapeDtypeStruct((C, C, n), jnp.float32),
        name=f"discrete_noise_posterior_c{C}",
    )(vT, q3, coef)


def kernel(Xt, Et, t):
    if Et.ndim == 3:
        Et = Et.reshape((-1, Et.shape[-1]))
    n_edges = Et.shape[0]

    a = _alpha(t).astype(jnp.float32)
    ab = _alpha(t - 1).astype(jnp.float32)

    def q3_of(C):
        o = ab / C
        d = o + (1.0 - ab)
        q = jnp.full((C, C), o, jnp.float32) + (d - o) * jnp.eye(
            C, dtype=jnp.float32)
        return q[:, :, None]

    def coef_of(C):
        return jnp.stack([a / C, 1.0 - a]).astype(jnp.float32)[:, None]

    Ep = _run(Et.T, q3_of(NE), coef_of(NE), NE, n_edges, _NB)
    Xp = _run(Xt.T, q3_of(NX), coef_of(NX), NX, N_NODES, N_NODES)
    return (jnp.transpose(Xp, (2, 0, 1)), jnp.transpose(Ep, (2, 0, 1)))


# speedup vs baseline: 40.0287x; 1.0381x over previous
import jax
import jax.numpy as jnp
from jax.experimental import pallas as pl

NX = 10
NE = 5
T = 1000
N_NODES = 1024

_NB = 65536


def _alpha(t):
    halfpi = 0.5 * jnp.pi
    s = 0.01
    return jnp.cos(halfpi * (t / T + s) / (1 + s))


def _posterior_body(x_ref, q_ref, c_ref, o_ref):
    C = x_ref.shape[0]
    nb = x_ref.shape[1]
    ca = c_ref[0, 0]
    cb = c_ref[1, 0]
    x = x_ref[...]
    s = jnp.sum(x, axis=0, keepdims=True)
    L = ca * s + cb * x
    den = jnp.where(L == 0.0, 1e-6, L)
    R = 1.0 / den
    q = jnp.broadcast_to(q_ref[...], (C, C, nb))
    o_ref[...] = q * R[:, None, :] * L[None, :, :]


def _run(vT, q3, coef, C, n, nb):
    grid = n // nb
    return pl.pallas_call(
        _posterior_body,
        grid=(grid,),
        in_specs=[
            pl.BlockSpec((C, nb), lambda i: (0, i)),
            pl.BlockSpec((C, C, 1), lambda i: (0, 0, 0)),
            pl.BlockSpec((2, 1), lambda i: (0, 0)),
        ],
        out_specs=pl.BlockSpec((C, C, nb), lambda i: (0, 0, i)),
        out_shape=jax.ShapeDtypeStruct((C, C, n), jnp.float32),
        name=f"discrete_noise_posterior_c{C}",
    )(vT, q3, coef)


def kernel(Xt, Et, t):
    if Et.ndim == 3:
        Et = Et.reshape((-1, Et.shape[-1]))
    n_edges = Et.shape[0]

    a = _alpha(t).astype(jnp.float32)
    ab = _alpha(t - 1).astype(jnp.float32)

    def q3_of(C):
        o = ab / C
        d = o + (1.0 - ab)
        q = jnp.full((C, C), o, jnp.float32) + (d - o) * jnp.eye(
            C, dtype=jnp.float32)
        return q[:, :, None]

    def coef_of(C):
        return jnp.stack([a / C, 1.0 - a]).astype(jnp.float32)[:, None]

    Ep = _run(Et.T, q3_of(NE), coef_of(NE), NE, n_edges, _NB)
    Xp = _run(Xt.T, q3_of(NX), coef_of(NX), NX, N_NODES, N_NODES)
    return (jnp.transpose(Xp, (2, 0, 1)), jnp.transpose(Ep, (2, 0, 1)))


# speedup vs baseline: 46.6189x; 1.1646x over previous
import jax
import jax.numpy as jnp
from jax.experimental import pallas as pl

NX = 10
NE = 5
T = 1000
N_NODES = 1024

_NB = 65536


def _alpha(t):
    halfpi = 0.5 * jnp.pi
    s = 0.01
    return jnp.cos(halfpi * (t / T + s) / (1 + s))


def _posterior_body(x_ref, q_ref, c_ref, o_ref):
    C = x_ref.shape[0]
    nb = x_ref.shape[1]
    ca = c_ref[0, 0]
    cb = c_ref[1, 0]
    x = x_ref[...]
    s = jnp.sum(x, axis=0, keepdims=True)
    L = ca * s + cb * x
    den = jnp.where(L == 0.0, 1e-6, L)
    R = 1.0 / den
    for j in range(C):
        rj = jnp.broadcast_to(R[j:j + 1, :], (C, nb))
        o_ref[j] = (q_ref[j] * L) * rj


def _run(vT, q3, coef, C, n, nb):
    grid = n // nb
    return pl.pallas_call(
        _posterior_body,
        grid=(grid,),
        in_specs=[
            pl.BlockSpec((C, nb), lambda i: (0, i)),
            pl.BlockSpec((C, C, 1), lambda i: (0, 0, 0)),
            pl.BlockSpec((2, 1), lambda i: (0, 0)),
        ],
        out_specs=pl.BlockSpec((C, C, nb), lambda i: (0, 0, i)),
        out_shape=jax.ShapeDtypeStruct((C, C, n), jnp.float32),
        name=f"discrete_noise_posterior_c{C}",
    )(vT, q3, coef)


def kernel(Xt, Et, t):
    if Et.ndim == 3:
        Et = Et.reshape((-1, Et.shape[-1]))
    n_edges = Et.shape[0]

    a = _alpha(t).astype(jnp.float32)
    ab = _alpha(t - 1).astype(jnp.float32)

    def q3_of(C):
        o = ab / C
        d = o + (1.0 - ab)
        q = jnp.full((C, C), o, jnp.float32) + (d - o) * jnp.eye(
            C, dtype=jnp.float32)
        return q[:, :, None]

    def coef_of(C):
        return jnp.stack([a / C, 1.0 - a]).astype(jnp.float32)[:, None]

    Ep = _run(Et.T, q3_of(NE), coef_of(NE), NE, n_edges, _NB)
    Xp = _run(Xt.T, q3_of(NX), coef_of(NX), NX, N_NODES, N_NODES)
    return (jnp.transpose(Xp, (2, 0, 1)), jnp.transpose(Ep, (2, 0, 1)))
